# baseline (speedup 1.0000x reference)
import functools

import jax
import jax.numpy as jnp
from jax import lax
from jax.experimental import pallas as pl
from jax.experimental.pallas import tpu as pltpu
from jax.experimental.pallas import tpu_sc as plsc

D_MODEL = 64
SCALE = float(D_MODEL) ** 0.5
NC, NS = 2, 16
NW = NC * NS
CHUNK = 128
NBUF = 4


def _scale_chunk(rows):
    def body(r, carry):
        for c in range(D_MODEL // 16):
            sl = (r, pl.ds(c * 16, 16))
            rows[sl] = rows[sl] * SCALE
        return carry
    lax.fori_loop(0, CHUNK, body, 0, unroll=8)


@functools.cache
def _make_kernel(n_rows):
    steps = n_rows // (NW * CHUNK)
    assert steps % NBUF == 0 and steps >= 2 * NBUF

    mesh = plsc.VectorSubcoreMesh(core_axis_name="c", subcore_axis_name="s")

    @functools.partial(
        pl.kernel,
        mesh=mesh,
        out_type=jax.ShapeDtypeStruct((n_rows, D_MODEL), jnp.float32),
        scratch_types=(
            [pltpu.VMEM((steps, CHUNK), jnp.int32)]
            + [pltpu.VMEM((CHUNK, D_MODEL), jnp.float32)] * NBUF
            + [pltpu.SemaphoreType.DMA] * (2 * NBUF)
        ),
        compiler_params=pltpu.CompilerParams(use_tc_tiling_on_sc=False),
    )
    def emb(idx_hbm, table_hbm, out_hbm, idx_v, r0, r1, r2, r3,
            g0, g1, g2, g3, s0, s1, s2, s3):
        bufs = (r0, r1, r2, r3)
        gsems = (g0, g1, g2, g3)
        ssems = (s0, s1, s2, s3)
        wid = lax.axis_index("s") * NC + lax.axis_index("c")
        idx_row0 = wid * steps
        out_base = idx_row0 * CHUNK

        pltpu.sync_copy(idx_hbm.at[pl.ds(idx_row0, steps)], idx_v)

        def g_start(s, b):
            pltpu.make_async_copy(
                table_hbm.at[idx_v.at[s]], bufs[b], gsems[b]).start()

        def g_wait(s, b):
            pltpu.make_async_copy(
                table_hbm.at[idx_v.at[s]], bufs[b], gsems[b]).wait()

        def st_start(s, b):
            pltpu.make_async_copy(
                bufs[b], out_hbm.at[pl.ds(out_base + s * CHUNK, CHUNK)],
                ssems[b]).start()

        def st_wait(s, b):
            pltpu.make_async_copy(
                bufs[b], out_hbm.at[pl.ds(out_base + s * CHUNK, CHUNK)],
                ssems[b]).wait()

        g_start(0, 0)
        g_start(1, 1)

        g_start(2, 2)
        g_wait(0, 0)
        _scale_chunk(bufs[0])
        st_start(0, 0)

        g_start(3, 3)
        g_wait(1, 1)
        _scale_chunk(bufs[1])
        st_start(1, 1)

        def body(i, carry):
            for k in range(NBUF):
                s = 2 + i * NBUF + k
                b = (2 + k) % NBUF
                b2 = k % NBUF
                st_wait(s - 2, b2)
                g_start(s + 2, b2)
                g_wait(s, b)
                _scale_chunk(bufs[b])
                st_start(s, b)
            return carry
        lax.fori_loop(0, (steps - 4) // NBUF, body, 0)

        g_wait(steps - 2, 2)
        _scale_chunk(bufs[2])
        st_start(steps - 2, 2)

        g_wait(steps - 1, 3)
        _scale_chunk(bufs[3])
        st_start(steps - 1, 3)

        st_wait(steps - 4, 0)
        st_wait(steps - 3, 1)
        st_wait(steps - 2, 2)
        st_wait(steps - 1, 3)

    return emb


def kernel(x, table):
    n = x.size
    idx = x.reshape(n // CHUNK, CHUNK).astype(jnp.int32)
    out = _make_kernel(n)(idx, table)
    return out.reshape(x.shape + (D_MODEL,))
